# baseline (speedup 1.0000x reference)
import functools

import jax
import jax.numpy as jnp
from jax import lax
from jax.experimental import pallas as pl
from jax.experimental.pallas import tpu as pltpu
from jax.experimental.pallas import tpu_sc as plsc

N = 100000
E = 625000
D = 128
N_PAD = 100096
CHUNK = 7168
NCHUNK = 14
LAST_CHUNK = N_PAD - 13 * CHUNK
ACC_ROWS = 7296
DUMMY = CHUNK
G = 128
NC = 2
NS = 16
EST = 19536
E_PAD = NC * NS * EST
CAP = 16 * EST + 16 * G
RB = 112

_MESH = plsc.VectorSubcoreMesh(
    core_axis_name="c", subcore_axis_name="s", num_cores=NC, num_subcores=NS)


def _bucketize_body(src_hbm, dst_hbm, srcb, dlocb, cnts,
                    ssrc, sdst, cb_src, cb_dloc, st16, cursors):
  k = lax.axis_index("c")
  s = lax.axis_index("s")
  t = k * NS + s
  for b in range(NCHUNK):
    cursors[b] = jnp.int32(0)
  plsc.subcore_barrier()
  base_e = t * EST
  pltpu.sync_copy(src_hbm.at[pl.ds(base_e, EST)], ssrc)
  pltpu.sync_copy(dst_hbm.at[pl.ds(base_e, EST)], sdst)
  lanes = lax.broadcasted_iota(jnp.int32, (16,), 0)
  for b in range(NCHUNK):
    lo = b * CHUNK
    hi = N_PAD if b == NCHUNK - 1 else (b + 1) * CHUNK

    def scan_body(v, cur, lo=lo, hi=hi):
      d16 = sdst[pl.ds(v * 16, 16)]
      gidx = base_e + v * 16 + lanes
      m = (gidx < E) & (d16 >= lo) & (d16 < hi)
      s16 = ssrc[pl.ds(v * 16, 16)]
      mi = m.astype(jnp.int32)
      pos = cur + plsc.cumsum(mi) - 1
      plsc.store_scatter(cb_src, [pos], s16, mask=m)
      plsc.store_scatter(cb_dloc, [pos], d16 - lo, mask=m)
      return cur + jnp.sum(mi)

    cur = lax.fori_loop(0, EST // 16, scan_body, jnp.int32(0))
    dummy_src = (base_e + lanes * 791) % N
    dummy_dloc = DUMMY + lanes
    for j in range(G // 16):
      plsc.store_scatter(cb_src, [cur + j * 16 + lanes], dummy_src)
      plsc.store_scatter(cb_dloc, [cur + j * 16 + lanes], dummy_dloc)
    flushlen = (cur + G - 1) & ~(G - 1)
    pos = plsc.fetch_and_add(cursors.at[b], flushlen, subcore_id=0)
    rbase = (k * NCHUNK + b) * CAP

    def flush_body(i, _, rbase=rbase, pos=pos):
      off = i * G
      dst_off = pl.multiple_of(rbase + pos + off, G)
      pltpu.sync_copy(cb_src.at[pl.ds(off, G)],
                      srcb.at[pl.ds(dst_off, G)])
      pltpu.sync_copy(cb_dloc.at[pl.ds(off, G)],
                      dlocb.at[pl.ds(dst_off, G)])
      return 0

    lax.fori_loop(0, flushlen >> 7, flush_body, 0)
  plsc.subcore_barrier()
  for b in range(NCHUNK):
    @pl.when(s == b)
    def _(b=b):
      cnt_b = plsc.fetch_and_add(cursors.at[b], 0, subcore_id=0)
      st16[...] = jnp.broadcast_to(cnt_b, (16,))
      coff = pl.multiple_of((k * NCHUNK + b) * 16, 16)
      pltpu.sync_copy(st16, cnts.at[pl.ds(coff, 16)])


def _step_body(h_in, srcb, dlocb, cnts, *refs, last):
  if last:
    (partials, acc_sh, cnt_sh, src_v, dloc_v, pay_v, zcnt,
     ones_v, cnt16v, acc_u, cnt_u, rsum_v, sem, sem_s) = refs
    h_out = None
  else:
    (h_out, acc_sh, cnt_sh, src_v, dloc_v, pay_v, zcnt,
     ones_v, cnt16v, acc_u, cnt_u, rsum_v, sem, sem_s) = refs
  k = lax.axis_index("c")
  s = lax.axis_index("s")
  zv = jnp.zeros((16,), jnp.float32)

  for j in range(G // 16):
    zcnt[pl.ds(j * 16, 16)] = zv
    ones_v[pl.ds(j * 16, 16)] = zv + 1.0
  if last:
    for c8 in range(8):
      rsum_v[pl.ds(c8 * 16, 16)] = zv

  def update_block(row0, nrows, c):
    row0 = pl.multiple_of(row0, 16)
    pltpu.sync_copy(acc_sh.at[pl.ds(row0, nrows)], acc_u.at[pl.ds(0, nrows)])
    pltpu.sync_copy(cnt_sh.at[pl.ds(row0, nrows)], cnt_u.at[pl.ds(0, nrows)])
    grow0 = c * CHUNK + row0
    pltpu.sync_copy(h_in.at[pl.ds(grow0, nrows)], pay_v.at[0, pl.ds(0, nrows)])

    def rbody(r, _):
      cs = plsc.load_gather(cnt_u, [jnp.broadcast_to(r, (16,)).astype(jnp.int32)])
      inv = 0.5 / jnp.maximum(cs, 1.0)
      for c8 in range(8):
        a = acc_u[r, pl.ds(c8 * 16, 16)]
        hh = pay_v[0, r, pl.ds(c8 * 16, 16)]
        o = hh * 0.5 + a * inv
        acc_u[r, pl.ds(c8 * 16, 16)] = o
        if last:
          rsum_v[pl.ds(c8 * 16, 16)] = (rsum_v[pl.ds(c8 * 16, 16)]
                                        + jnp.maximum(o, 0.0))
      return 0

    lax.fori_loop(0, nrows, rbody, 0)
    if not last:
      pltpu.sync_copy(acc_u.at[pl.ds(0, nrows)], h_out.at[pl.ds(grow0, nrows)])

  for c in range(NCHUNK):
    csize = CHUNK if c < NCHUNK - 1 else LAST_CHUNK
    rpt = csize // 16

    @pl.when(k == (0 if c < 7 else 1))
    def _(c=c, csize=csize, rpt=rpt):
      def zfill(r, _):
        for c8 in range(8):
          pay_v[0, r, pl.ds(c8 * 16, 16)] = zv
        return 0

      lax.fori_loop(0, G, zfill, 0)
      nz = ACC_ROWS // G
      for jj in range((nz + 15) // 16):
        @pl.when(s + 16 * jj < nz)
        def _(jj=jj):
          off = (s + 16 * jj) * G
          pltpu.sync_copy(pay_v.at[0], acc_sh.at[pl.ds(off, G)])
          pltpu.sync_copy(zcnt, cnt_sh.at[pl.ds(off, G)])
      plsc.subcore_barrier()
      for kp in range(NC):
        pltpu.sync_copy(cnts.at[pl.ds((kp * NCHUNK + c) * 16, 16)], cnt16v)
        trips = jnp.max(cnt16v[...]) // G
        my_n = jnp.maximum((trips - s + NS - 1) // NS, 0)
        rbase = (kp * NCHUNK + c) * CAP

        def blk_off(j, rbase=rbase):
          return pl.multiple_of(rbase + (s + j * NS) * G, G)

        @pl.when(my_n > 0)
        def _(blk_off=blk_off):
          off = blk_off(0)
          pltpu.sync_copy(srcb.at[pl.ds(off, G)], src_v.at[0])
          pltpu.sync_copy(dlocb.at[pl.ds(off, G)], dloc_v.at[0])
          pltpu.async_copy(h_in.at[src_v.at[0]], pay_v.at[0], sem)

        def ebody2(ii, _, blk_off=blk_off, my_n=my_n):
          for b2 in range(2):
            j = ii * 2 + b2
            nb = (b2 + 1) % 2

            @pl.when(j < my_n)
            def _(j=j, b2=b2):
              pltpu.make_async_copy(
                  h_in.at[src_v.at[b2]], pay_v.at[b2], sem).wait()
              pltpu.async_copy(pay_v.at[b2], acc_sh.at[dloc_v.at[b2]], sem_s,
                               add=True)
              pltpu.async_copy(ones_v, cnt_sh.at[dloc_v.at[b2]], sem_s,
                               add=True)

            @pl.when(j + 1 < my_n)
            def _(j=j, nb=nb):
              @pl.when(j >= 1)
              def _(nb=nb):
                pltpu.make_async_copy(
                    pay_v.at[nb], acc_sh.at[dloc_v.at[nb]], sem_s).wait()
                pltpu.make_async_copy(
                    ones_v, cnt_sh.at[dloc_v.at[nb]], sem_s).wait()
              off = blk_off(j + 1)
              pltpu.sync_copy(srcb.at[pl.ds(off, G)], src_v.at[nb])
              pltpu.sync_copy(dlocb.at[pl.ds(off, G)], dloc_v.at[nb])
              pltpu.async_copy(h_in.at[src_v.at[nb]], pay_v.at[nb], sem)
          return 0

        lax.fori_loop(0, (my_n + 1) // 2, ebody2, 0)
        @pl.when(my_n >= 1)
        def _():
          pltpu.make_async_copy(
              pay_v.at[0], acc_sh.at[dloc_v.at[0]], sem_s).wait()
          pltpu.make_async_copy(ones_v, cnt_sh.at[dloc_v.at[0]], sem_s).wait()

        @pl.when(my_n >= 2)
        def _():
          pltpu.make_async_copy(
              pay_v.at[1], acc_sh.at[dloc_v.at[1]], sem_s).wait()
          pltpu.make_async_copy(ones_v, cnt_sh.at[dloc_v.at[1]], sem_s).wait()
      plsc.subcore_barrier()
      nb_full = rpt // RB
      rem = rpt - nb_full * RB

      def ubody(i, _, rpt=rpt, c=c):
        update_block(s * rpt + i * RB, RB, c)
        return 0

      lax.fori_loop(0, nb_full, ubody, 0)
      if rem:
        update_block(s * rpt + nb_full * RB, rem, c)

  if last:
    plsc.subcore_barrier()
    pltpu.sync_copy(rsum_v, partials.at[k * NS + s])


def _sc_bucketize(src_pad, dst_pad):
  return pl.kernel(
      _bucketize_body,
      out_type=[
          jax.ShapeDtypeStruct((NC * NCHUNK * CAP,), jnp.int32),
          jax.ShapeDtypeStruct((NC * NCHUNK * CAP,), jnp.int32),
          jax.ShapeDtypeStruct((NC * NCHUNK * 16,), jnp.int32),
      ],
      mesh=_MESH,
      compiler_params=pltpu.CompilerParams(needs_layout_passes=False),
      scratch_types=[
          pltpu.VMEM((EST,), jnp.int32),
          pltpu.VMEM((EST,), jnp.int32),
          pltpu.VMEM((EST + G,), jnp.int32),
          pltpu.VMEM((EST + G,), jnp.int32),
          pltpu.VMEM((16,), jnp.int32),
          pltpu.SMEM((NCHUNK,), jnp.int32),
      ],
      name="mpnn_bucketize",
  )(src_pad, dst_pad)


def _sc_step(h, srcb, dlocb, cnts, last):
  if last:
    out_type = jax.ShapeDtypeStruct((NC * NS, D), jnp.float32)
  else:
    out_type = jax.ShapeDtypeStruct((N_PAD, D), jnp.float32)
  return pl.kernel(
      functools.partial(_step_body, last=last),
      out_type=out_type,
      mesh=_MESH,
      compiler_params=pltpu.CompilerParams(needs_layout_passes=False),
      scratch_types=[
          pltpu.VMEM_SHARED((ACC_ROWS, D), jnp.float32),
          pltpu.VMEM_SHARED((ACC_ROWS,), jnp.float32),
          pltpu.VMEM((2, G), jnp.int32),
          pltpu.VMEM((2, G), jnp.int32),
          pltpu.VMEM((2, G, D), jnp.float32),
          pltpu.VMEM((G,), jnp.float32),
          pltpu.VMEM((G,), jnp.float32),
          pltpu.VMEM((16,), jnp.int32),
          pltpu.VMEM((RB, D), jnp.float32),
          pltpu.VMEM((RB,), jnp.float32),
          pltpu.VMEM((D,), jnp.float32),
          pltpu.SemaphoreType.DMA,
          pltpu.SemaphoreType.DMA,
      ],
      name="mpnn_step",
  )(h, srcb, dlocb, cnts)


def _h0_kernel(x_ref, w_ref, b_ref, o_ref):
  i = pl.program_id(0)
  h = lax.dot_general(x_ref[...], w_ref[...], (((1,), (1,)), ((), ())),
                      preferred_element_type=jnp.float32)
  h = jnp.maximum(h + b_ref[...], 0.0)
  rows = i * _BR + lax.broadcasted_iota(jnp.int32, o_ref.shape, 0)
  o_ref[...] = jnp.where(rows < N, h, 0.0)


_BR = 2176


def _tc_input_mlp(x_pad, W_in, b_in):
  return pl.pallas_call(
      _h0_kernel,
      grid=(N_PAD // _BR,),
      in_specs=[
          pl.BlockSpec((_BR, D), lambda i: (i, 0)),
          pl.BlockSpec((D, D), lambda i: (0, 0)),
          pl.BlockSpec((1, D), lambda i: (0, 0)),
      ],
      out_specs=pl.BlockSpec((_BR, D), lambda i: (i, 0)),
      out_shape=jax.ShapeDtypeStruct((N_PAD, D), jnp.float32),
  )(x_pad, W_in, b_in.reshape(1, D))


def _ro_kernel(p_ref, wro_ref, bro_ref, wp_ref, bp_ref, o_ref):
  m = jnp.sum(p_ref[...], axis=0, keepdims=True) * (1.0 / N)
  z = lax.dot_general(m, wro_ref[...], (((1,), (1,)), ((), ())),
                      preferred_element_type=jnp.float32)
  z = jnp.maximum(z + bro_ref[...], 0.0)
  zb = jnp.broadcast_to(z, (8, D))
  o = lax.dot_general(zb, wp_ref[...], (((1,), (1,)), ((), ())),
                      preferred_element_type=jnp.float32)
  o_ref[...] = o + bp_ref[...]


def _tc_readout(partials, W_ro, b_ro, W_pred, b_pred):
  wp8 = jnp.broadcast_to(W_pred, (8, D))
  bp8 = jnp.broadcast_to(b_pred.reshape(1, 1), (8, 8))
  out = pl.pallas_call(
      _ro_kernel,
      out_shape=jax.ShapeDtypeStruct((8, 8), jnp.float32),
  )(partials, W_ro, b_ro.reshape(1, D), wp8, bp8)
  return out[0:1, 0]


def kernel(x, edge_index, W_in, b_in, W_ro, b_ro, W_pred, b_pred):
  ei = edge_index.astype(jnp.int32)
  src_pad = jnp.pad(ei[0], (0, E_PAD - E))
  dst_pad = jnp.pad(ei[1], (0, E_PAD - E))
  x_pad = jnp.pad(x, ((0, N_PAD - N), (0, 0)))
  h = _tc_input_mlp(x_pad, W_in, b_in)
  srcb, dlocb, cnts = _sc_bucketize(src_pad, dst_pad)
  for step in range(3):
    out = _sc_step(h, srcb, dlocb, cnts, last=(step == 2))
    h = out
  partials = out
  return _tc_readout(partials, W_ro, b_ro, W_pred, b_pred)


# speedup vs baseline: 1.1641x; 1.1641x over previous
import functools

import jax
import jax.numpy as jnp
from jax import lax
from jax.experimental import pallas as pl
from jax.experimental.pallas import tpu as pltpu
from jax.experimental.pallas import tpu_sc as plsc

N = 100000
E = 625000
D = 128
N_PAD = 100096
CHUNK = 7168
NCHUNK = 14
LAST_CHUNK = N_PAD - 13 * CHUNK
ACC_ROWS = 7296
DUMMY = CHUNK
G = 128
NC = 2
NS = 16
EST = 19536
E_PAD = NC * NS * EST
CAP = 16 * EST + 16 * G
RB = 112

_MESH = plsc.VectorSubcoreMesh(
    core_axis_name="c", subcore_axis_name="s", num_cores=NC, num_subcores=NS)


def _bucketize_body(src_hbm, dst_hbm, srcb, dlocb, cnts,
                    ssrc, sdst, cb_src, cb_dloc, st16, cursors):
  k = lax.axis_index("c")
  s = lax.axis_index("s")
  t = k * NS + s
  for b in range(NCHUNK):
    cursors[b] = jnp.int32(0)
  plsc.subcore_barrier()
  base_e = t * EST
  pltpu.sync_copy(src_hbm.at[pl.ds(base_e, EST)], ssrc)
  pltpu.sync_copy(dst_hbm.at[pl.ds(base_e, EST)], sdst)
  lanes = lax.broadcasted_iota(jnp.int32, (16,), 0)
  for b in range(NCHUNK):
    lo = b * CHUNK
    hi = N_PAD if b == NCHUNK - 1 else (b + 1) * CHUNK

    def scan_body(v, cur, lo=lo, hi=hi):
      d16 = sdst[pl.ds(v * 16, 16)]
      gidx = base_e + v * 16 + lanes
      m = (gidx < E) & (d16 >= lo) & (d16 < hi)
      s16 = ssrc[pl.ds(v * 16, 16)]
      mi = m.astype(jnp.int32)
      pos = cur + plsc.cumsum(mi) - 1
      plsc.store_scatter(cb_src, [pos], s16, mask=m)
      plsc.store_scatter(cb_dloc, [pos], d16 - lo, mask=m)
      return cur + jnp.sum(mi)

    cur = lax.fori_loop(0, EST // 16, scan_body, jnp.int32(0))
    dummy_src = (base_e + lanes * 791) % N
    dummy_dloc = DUMMY + lanes
    for j in range(G // 16):
      plsc.store_scatter(cb_src, [cur + j * 16 + lanes], dummy_src)
      plsc.store_scatter(cb_dloc, [cur + j * 16 + lanes], dummy_dloc)
    flushlen = (cur + G - 1) & ~(G - 1)
    pos = plsc.fetch_and_add(cursors.at[b], flushlen, subcore_id=0)
    rbase = (k * NCHUNK + b) * CAP

    def flush_body(i, _, rbase=rbase, pos=pos):
      off = i * G
      dst_off = pl.multiple_of(rbase + pos + off, G)
      pltpu.sync_copy(cb_src.at[pl.ds(off, G)],
                      srcb.at[pl.ds(dst_off, G)])
      pltpu.sync_copy(cb_dloc.at[pl.ds(off, G)],
                      dlocb.at[pl.ds(dst_off, G)])
      return 0

    lax.fori_loop(0, flushlen >> 7, flush_body, 0)
  plsc.subcore_barrier()
  for b in range(NCHUNK):
    @pl.when(s == b)
    def _(b=b):
      cnt_b = plsc.fetch_and_add(cursors.at[b], 0, subcore_id=0)
      st16[...] = jnp.broadcast_to(cnt_b, (16,))
      coff = pl.multiple_of((k * NCHUNK + b) * 16, 16)
      pltpu.sync_copy(st16, cnts.at[pl.ds(coff, 16)])


def _step_body(*args, mode):
  last = mode == 2
  if mode == 0:
    (h_in, srcb, dlocb, cnts, h_out, cnt_out, *scr) = args
    cinv = partials = None
  elif mode == 1:
    (h_in, srcb, dlocb, cnts, cinv, h_out, *scr) = args
    cnt_out = partials = None
  else:
    (h_in, srcb, dlocb, cnts, cinv, partials, *scr) = args
    h_out = cnt_out = None
  (acc_sh, cnt_sh, src_v, dloc_v, pay_v, zcnt,
   ones_v, cnt16v, acc_u, cnt_u, rsum_v, sem, sem_s) = scr
  k = lax.axis_index("c")
  s = lax.axis_index("s")
  zv = jnp.zeros((16,), jnp.float32)

  for j in range(G // 16):
    zcnt[pl.ds(j * 16, 16)] = zv
    ones_v[pl.ds(j * 16, 16)] = zv + 1.0
  if last:
    for c8 in range(8):
      rsum_v[pl.ds(c8 * 16, 16)] = zv

  def update_block(row0, nrows, c):
    row0 = pl.multiple_of(row0, 16)
    pltpu.sync_copy(acc_sh.at[pl.ds(row0, nrows)], acc_u.at[pl.ds(0, nrows)])
    grow0 = c * CHUNK + row0
    if mode == 0:
      pltpu.sync_copy(cnt_sh.at[pl.ds(row0, nrows)], cnt_u.at[pl.ds(0, nrows)])
    else:
      pltpu.sync_copy(cinv.at[pl.ds(grow0, nrows)], cnt_u.at[pl.ds(0, nrows)])
    pltpu.sync_copy(h_in.at[pl.ds(grow0, nrows)], pay_v.at[0, pl.ds(0, nrows)])
    if mode == 0:
      for g in range(nrows // 16):
        cg = cnt_u[pl.ds(g * 16, 16)]
        cnt_u[pl.ds(g * 16, 16)] = 0.5 / jnp.maximum(cg, 1.0)
      pltpu.sync_copy(cnt_u.at[pl.ds(0, nrows)], cnt_out.at[pl.ds(grow0, nrows)])

    def rbody(r, _):
      inv = plsc.load_gather(cnt_u, [jnp.broadcast_to(r, (16,)).astype(jnp.int32)])
      for c8 in range(8):
        a = acc_u[r, pl.ds(c8 * 16, 16)]
        hh = pay_v[0, r, pl.ds(c8 * 16, 16)]
        o = hh * 0.5 + a * inv
        acc_u[r, pl.ds(c8 * 16, 16)] = o
        if last:
          rsum_v[pl.ds(c8 * 16, 16)] = (rsum_v[pl.ds(c8 * 16, 16)]
                                        + jnp.maximum(o, 0.0))
      return 0

    lax.fori_loop(0, nrows, rbody, 0)
    if not last:
      pltpu.sync_copy(acc_u.at[pl.ds(0, nrows)], h_out.at[pl.ds(grow0, nrows)])

  for c in range(NCHUNK):
    csize = CHUNK if c < NCHUNK - 1 else LAST_CHUNK
    rpt = csize // 16

    @pl.when(k == (0 if c < 7 else 1))
    def _(c=c, csize=csize, rpt=rpt):
      def zfill(r, _):
        for c8 in range(8):
          pay_v[0, r, pl.ds(c8 * 16, 16)] = zv
        return 0

      lax.fori_loop(0, G, zfill, 0)
      nz = ACC_ROWS // G
      for jj in range((nz + 15) // 16):
        @pl.when(s + 16 * jj < nz)
        def _(jj=jj):
          off = (s + 16 * jj) * G
          pltpu.sync_copy(pay_v.at[0], acc_sh.at[pl.ds(off, G)])
          if mode == 0:
            pltpu.sync_copy(zcnt, cnt_sh.at[pl.ds(off, G)])
      plsc.subcore_barrier()
      for kp in range(NC):
        pltpu.sync_copy(cnts.at[pl.ds((kp * NCHUNK + c) * 16, 16)], cnt16v)
        trips = jnp.max(cnt16v[...]) // G
        my_n = jnp.maximum((trips - s + NS - 1) // NS, 0)
        rbase = (kp * NCHUNK + c) * CAP

        def blk_off(j, rbase=rbase):
          return pl.multiple_of(rbase + (s + j * NS) * G, G)

        @pl.when(my_n > 0)
        def _(blk_off=blk_off):
          off = blk_off(0)
          pltpu.sync_copy(srcb.at[pl.ds(off, G)], src_v.at[0])
          pltpu.sync_copy(dlocb.at[pl.ds(off, G)], dloc_v.at[0])
          pltpu.async_copy(h_in.at[src_v.at[0]], pay_v.at[0], sem)

        def ebody2(ii, _, blk_off=blk_off, my_n=my_n):
          for b2 in range(2):
            j = ii * 2 + b2
            nb = (b2 + 1) % 2

            @pl.when(j + 1 < my_n)
            def _(j=j, nb=nb):
              off = blk_off(j + 1)
              pltpu.sync_copy(srcb.at[pl.ds(off, G)], src_v.at[nb])
              pltpu.sync_copy(dlocb.at[pl.ds(off, G)], dloc_v.at[nb])
              pltpu.async_copy(h_in.at[src_v.at[nb]], pay_v.at[nb], sem)

            @pl.when(j < my_n)
            def _(j=j, b2=b2):
              pltpu.make_async_copy(
                  h_in.at[src_v.at[b2]], pay_v.at[b2], sem).wait()
              pltpu.sync_copy(pay_v.at[b2], acc_sh.at[dloc_v.at[b2]], add=True)
              if mode == 0:
                pltpu.sync_copy(ones_v, cnt_sh.at[dloc_v.at[b2]], add=True)
          return 0

        lax.fori_loop(0, (my_n + 1) // 2, ebody2, 0)
      plsc.subcore_barrier()
      nb_full = rpt // RB
      rem = rpt - nb_full * RB

      def ubody(i, _, rpt=rpt, c=c):
        update_block(s * rpt + i * RB, RB, c)
        return 0

      lax.fori_loop(0, nb_full, ubody, 0)
      if rem:
        update_block(s * rpt + nb_full * RB, rem, c)

  if last:
    plsc.subcore_barrier()
    pltpu.sync_copy(rsum_v, partials.at[k * NS + s])


def _sc_bucketize(src_pad, dst_pad):
  return pl.kernel(
      _bucketize_body,
      out_type=[
          jax.ShapeDtypeStruct((NC * NCHUNK * CAP,), jnp.int32),
          jax.ShapeDtypeStruct((NC * NCHUNK * CAP,), jnp.int32),
          jax.ShapeDtypeStruct((NC * NCHUNK * 16,), jnp.int32),
      ],
      mesh=_MESH,
      compiler_params=pltpu.CompilerParams(needs_layout_passes=False),
      scratch_types=[
          pltpu.VMEM((EST,), jnp.int32),
          pltpu.VMEM((EST,), jnp.int32),
          pltpu.VMEM((EST + G,), jnp.int32),
          pltpu.VMEM((EST + G,), jnp.int32),
          pltpu.VMEM((16,), jnp.int32),
          pltpu.SMEM((NCHUNK,), jnp.int32),
      ],
      name="mpnn_bucketize",
  )(src_pad, dst_pad)


def _sc_step(h, srcb, dlocb, cnts, mode, cinv=None):
  if mode == 0:
    out_type = [jax.ShapeDtypeStruct((N_PAD, D), jnp.float32),
                jax.ShapeDtypeStruct((N_PAD,), jnp.float32)]
  elif mode == 1:
    out_type = jax.ShapeDtypeStruct((N_PAD, D), jnp.float32)
  else:
    out_type = jax.ShapeDtypeStruct((NC * NS, D), jnp.float32)
  return pl.kernel(
      functools.partial(_step_body, mode=mode),
      out_type=out_type,
      mesh=_MESH,
      compiler_params=pltpu.CompilerParams(needs_layout_passes=False),
      scratch_types=[
          pltpu.VMEM_SHARED((ACC_ROWS, D), jnp.float32),
          pltpu.VMEM_SHARED((ACC_ROWS,), jnp.float32),
          pltpu.VMEM((2, G), jnp.int32),
          pltpu.VMEM((2, G), jnp.int32),
          pltpu.VMEM((2, G, D), jnp.float32),
          pltpu.VMEM((G,), jnp.float32),
          pltpu.VMEM((G,), jnp.float32),
          pltpu.VMEM((16,), jnp.int32),
          pltpu.VMEM((RB, D), jnp.float32),
          pltpu.VMEM((RB,), jnp.float32),
          pltpu.VMEM((D,), jnp.float32),
          pltpu.SemaphoreType.DMA,
          pltpu.SemaphoreType.DMA,
      ],
      name="mpnn_step",
  )(*((h, srcb, dlocb, cnts) if mode == 0 else (h, srcb, dlocb, cnts, cinv)))


def _h0_kernel(x_ref, w_ref, b_ref, o_ref):
  i = pl.program_id(0)
  h = lax.dot_general(x_ref[...], w_ref[...], (((1,), (1,)), ((), ())),
                      preferred_element_type=jnp.float32)
  h = jnp.maximum(h + b_ref[...], 0.0)
  rows = i * _BR + lax.broadcasted_iota(jnp.int32, o_ref.shape, 0)
  o_ref[...] = jnp.where(rows < N, h, 0.0)


_BR = 2176


def _tc_input_mlp(x_pad, W_in, b_in):
  return pl.pallas_call(
      _h0_kernel,
      grid=(N_PAD // _BR,),
      in_specs=[
          pl.BlockSpec((_BR, D), lambda i: (i, 0)),
          pl.BlockSpec((D, D), lambda i: (0, 0)),
          pl.BlockSpec((1, D), lambda i: (0, 0)),
      ],
      out_specs=pl.BlockSpec((_BR, D), lambda i: (i, 0)),
      out_shape=jax.ShapeDtypeStruct((N_PAD, D), jnp.float32),
  )(x_pad, W_in, b_in.reshape(1, D))


def _ro_kernel(p_ref, wro_ref, bro_ref, wp_ref, bp_ref, o_ref):
  m = jnp.sum(p_ref[...], axis=0, keepdims=True) * (1.0 / N)
  z = lax.dot_general(m, wro_ref[...], (((1,), (1,)), ((), ())),
                      preferred_element_type=jnp.float32)
  z = jnp.maximum(z + bro_ref[...], 0.0)
  zb = jnp.broadcast_to(z, (8, D))
  o = lax.dot_general(zb, wp_ref[...], (((1,), (1,)), ((), ())),
                      preferred_element_type=jnp.float32)
  o_ref[...] = o + bp_ref[...]


def _tc_readout(partials, W_ro, b_ro, W_pred, b_pred):
  wp8 = jnp.broadcast_to(W_pred, (8, D))
  bp8 = jnp.broadcast_to(b_pred.reshape(1, 1), (8, 8))
  out = pl.pallas_call(
      _ro_kernel,
      out_shape=jax.ShapeDtypeStruct((8, 8), jnp.float32),
  )(partials, W_ro, b_ro.reshape(1, D), wp8, bp8)
  return out[0:1, 0]


def kernel(x, edge_index, W_in, b_in, W_ro, b_ro, W_pred, b_pred):
  ei = edge_index.astype(jnp.int32)
  src_pad = jnp.pad(ei[0], (0, E_PAD - E))
  dst_pad = jnp.pad(ei[1], (0, E_PAD - E))
  x_pad = jnp.pad(x, ((0, N_PAD - N), (0, 0)))
  h = _tc_input_mlp(x_pad, W_in, b_in)
  srcb, dlocb, cnts = _sc_bucketize(src_pad, dst_pad)
  h, cinv = _sc_step(h, srcb, dlocb, cnts, mode=0)
  h = _sc_step(h, srcb, dlocb, cnts, mode=1, cinv=cinv)
  partials = _sc_step(h, srcb, dlocb, cnts, mode=2, cinv=cinv)
  return _tc_readout(partials, W_ro, b_ro, W_pred, b_pred)
